# Initial kernel scaffold; baseline (speedup 1.0000x reference)
import jax
import jax.numpy as jnp
from jax.experimental import pallas as pl


def kernel(X, N, targets, sources, degree, attn_kernel_self, attn_kernel_adjc):
    raise NotImplementedError("write your pallas kernel here")


# speedup vs baseline: 1.5754x; 1.5754x over previous
import functools

import jax
import jax.numpy as jnp
from jax.experimental import pallas as pl
from jax.experimental.pallas import tpu as pltpu


def _logits_body(x_ref, w_ref, o_ref):
    o_ref[...] = jnp.dot(x_ref[...], w_ref[...],
                         preferred_element_type=jnp.float32,
                         precision=jax.lax.Precision.HIGHEST)


def _node_logits(X, attn_kernel_self, attn_kernel_adjc):
    _, N, H, D = X.shape
    X2 = X.reshape(N, H * D)
    ws = attn_kernel_self[:, :, 0]
    wa = attn_kernel_adjc[:, :, 0]
    eye = jnp.eye(H, dtype=X.dtype)
    Wself = (eye[:, None, :] * ws.T[:, :, None]).reshape(H * D, H)
    Wadjc = (eye[:, None, :] * wa.T[:, :, None]).reshape(H * D, H)
    W = jnp.concatenate([Wself, Wadjc], axis=1)
    BN = 1000
    grid = (N // BN,)
    return pl.pallas_call(
        _logits_body,
        grid=grid,
        in_specs=[
            pl.BlockSpec((BN, H * D), lambda i: (i, 0)),
            pl.BlockSpec((H * D, 2 * H), lambda i: (0, 0)),
        ],
        out_specs=pl.BlockSpec((BN, 2 * H), lambda i: (i, 0)),
        out_shape=jax.ShapeDtypeStruct((N, 2 * H), jnp.float32),
    )(X2, W)


def kernel(X, N, targets, sources, degree, attn_kernel_self, attn_kernel_adjc):
    del degree
    _, N_static, H, D = X.shape
    E = targets.shape[1]

    logits = _node_logits(X, attn_kernel_self, attn_kernel_adjc)
    ls = logits[:, :H]
    la = logits[:, H:]

    t = targets[0]
    s = sources[0]
    e = ls[t] + la[s]
    e = jnp.where(e >= 0, e, 0.2 * e)
    m = jax.ops.segment_max(e, t, num_segments=N_static)
    out = jnp.exp(e - m[t])
    keep = jax.random.bernoulli(jax.random.key(42), 0.5, (1, E, H))
    scale = jnp.float32(2.0 / (1.0 + 1e-9))
    out = jnp.where(keep[0], out * scale, 0.0)
    return out[None, ..., None]
